# baseline (speedup 1.0000x reference)
import functools

import jax
import jax.numpy as jnp
from jax import lax
from jax.experimental import pallas as pl
from jax.experimental.pallas import tpu as pltpu
from jax.experimental.pallas import tpu_sc as plsc

SC_B = 1


def _add_body(x_ref, pos_ref, out_ref):
    out_ref[...] = x_ref[...] + pos_ref[...]


def _tc_part(x, pos_table, nb):
    B, T, D = x.shape
    BT = 512
    grid = (T // BT, nb)
    return pl.pallas_call(
        _add_body,
        grid=grid,
        in_specs=[
            pl.BlockSpec((1, BT, D), lambda t, b: (b, t, 0)),
            pl.BlockSpec((BT, D), lambda t, b: (t, 0)),
        ],
        out_specs=pl.BlockSpec((1, BT, D), lambda t, b: (b, t, 0)),
        out_shape=jax.ShapeDtypeStruct((nb, T, D), x.dtype),
    )(x, pos_table)


def _sc_part(x, pos_table, row_base, nrows):
    B, T, D = x.shape
    NW = 32
    RPW = nrows // NW
    R = 16
    NSTEPS = RPW // R
    NB = 2

    x_flat = x.reshape(B * T, D)

    mesh = plsc.VectorSubcoreMesh(core_axis_name="c", subcore_axis_name="s")

    @functools.partial(
        pl.kernel,
        mesh=mesh,
        out_type=jax.ShapeDtypeStruct((nrows, D), jnp.float32),
        scratch_types=[
            pltpu.VMEM((NB, R, D), jnp.float32),
            pltpu.VMEM((NB, R, D), jnp.float32),
            pltpu.SemaphoreType.DMA((NB,)),
            pltpu.SemaphoreType.DMA((NB,)),
            pltpu.SemaphoreType.DMA((NB,)),
        ],
    )
    def sc_add(x_hbm, pos_hbm, out_hbm, x_buf, pos_buf, xsem, psem, osem):
        c = lax.axis_index("c")
        s = lax.axis_index("s")
        wid = c * 16 + s
        orow0 = wid * RPW
        prow0 = lax.rem(row_base + orow0, T)

        def orow(k):
            return pl.multiple_of(orow0 + k * R, R)

        def xrow(k):
            return pl.multiple_of(row_base + orow0 + k * R, R)

        def prow(k):
            return pl.multiple_of(prow0 + k * R, R)

        def start_loads(k):
            p = k % NB
            dx = pltpu.async_copy(
                x_hbm.at[pl.ds(xrow(k), R)], x_buf.at[p], xsem.at[p])
            dp = pltpu.async_copy(
                pos_hbm.at[pl.ds(prow(k), R)], pos_buf.at[p], psem.at[p])
            return dx, dp

        loads = {0: start_loads(0)}
        stores = {}
        for k in range(NSTEPS):
            p = k % NB
            if k + 1 < NSTEPS:
                if k - 1 in stores:
                    stores.pop(k - 1).wait()
                loads[k + 1] = start_loads(k + 1)
            dx, dp = loads.pop(k)
            dx.wait()
            dp.wait()

            @plsc.parallel_loop(0, R * D, step=16, unroll=8)
            def _(i):
                r = i // D
                d0 = pl.multiple_of(i % D, 16)
                sl = pl.ds(d0, 16)
                plsc.addupdate(pos_buf.at[p, r].at[sl], x_buf[p, r, sl])

            stores[k] = pltpu.async_copy(
                pos_buf.at[p], out_hbm.at[pl.ds(orow(k), R)], osem.at[p])
        for k in sorted(stores):
            stores.pop(k).wait()

    return sc_add(x_flat, pos_table)


def kernel(x, pos_table):
    B, T, D = x.shape
    tc_b = B - SC_B
    out_sc = _sc_part(x, pos_table, tc_b * T, SC_B * T)
    out_tc = _tc_part(x, pos_table, tc_b)
    return jnp.concatenate([out_tc, out_sc.reshape(SC_B, T, D)], axis=0)


# speedup vs baseline: 2.3073x; 2.3073x over previous
import jax
import jax.numpy as jnp
from jax.experimental import pallas as pl


def _add_body(x_ref, pos_ref, out_ref):
    out_ref[...] = x_ref[...] + pos_ref[...]


def kernel(x, pos_table):
    B, T, D = x.shape
    BT = 1024
    grid = (T // BT, B)
    return pl.pallas_call(
        _add_body,
        grid=grid,
        in_specs=[
            pl.BlockSpec((1, BT, D), lambda t, b: (b, t, 0)),
            pl.BlockSpec((BT, D), lambda t, b: (t, 0)),
        ],
        out_specs=pl.BlockSpec((1, BT, D), lambda t, b: (b, t, 0)),
        out_shape=jax.ShapeDtypeStruct((B, T, D), x.dtype),
    )(x, pos_table)


# speedup vs baseline: 2.4624x; 1.0672x over previous
import jax
import jax.numpy as jnp
from jax.experimental import pallas as pl


def _add_body(x_ref, pos_ref, out_ref):
    out_ref[...] = x_ref[...] + pos_ref[...]


def kernel(x, pos_table):
    B, T, D = x.shape
    BT = 2048
    grid = (T // BT, B)
    return pl.pallas_call(
        _add_body,
        grid=grid,
        in_specs=[
            pl.BlockSpec((1, BT, D), lambda t, b: (b, t, 0)),
            pl.BlockSpec((BT, D), lambda t, b: (t, 0)),
        ],
        out_specs=pl.BlockSpec((1, BT, D), lambda t, b: (b, t, 0)),
        out_shape=jax.ShapeDtypeStruct((B, T, D), x.dtype),
    )(x, pos_table)
